# baseline (device time: 77966 ns/iter reference)
import numpy as np
import jax
import jax.numpy as jnp
from jax import lax
from jax.experimental import pallas as pl
from jax.experimental.pallas import tpu as pltpu

N_DEV = 8

_COORDS = {0: (0, 0, 0), 1: (1, 0, 0), 2: (1, 1, 0), 3: (0, 1, 0),
           4: (0, 0, 1), 5: (1, 0, 1), 6: (1, 1, 1), 7: (0, 1, 1)}
_INV = {v: k for k, v in _COORDS.items()}

_NBR = np.zeros((N_DEV, 3), dtype=np.int32)
for _u, (_x, _y, _z) in _COORDS.items():
    _NBR[_u, 0] = _INV[(1 - _x, _y, _z)]
    _NBR[_u, 1] = _INV[(_x, 1 - _y, _z)]
    _NBR[_u, 2] = _INV[(_x, _y, 1 - _z)]

_STREAMS = [
    ((0, 1, 2), 0, 192),
    ((1, 2, 0), 192, 160),
    ((2, 0, 1), 352, 160),
]


def kernel(x, w_mat, scale_x, scale_w):
    m_per, k = x.shape
    _, n_per = w_mat.shape
    M = N_DEV * m_per

    nbr_tab = jnp.asarray(_NBR)

    def body(x_ref, w_ref, sx_ref, sw_ref, nbr_ref, out_ref,
             c0_ref, c1_ref, c2_ref, send_sems, recv_sems,
             xv_ref, wv_ref, yv_ref, in_sems, out_sems):
        comm = [c0_ref, c1_ref, c2_ref]
        my = lax.axis_index("i")

        x_copy = pltpu.make_async_copy(x_ref, xv_ref, in_sems.at[0])
        w_copy = pltpu.make_async_copy(w_ref, wv_ref, in_sems.at[1])
        x_copy.start()
        w_copy.start()

        ids = []
        for (d1, d2, d3), _, _ in _STREAMS:
            n1 = nbr_ref[my, d1]
            n2 = nbr_ref[my, d2]
            n3 = nbr_ref[my, d3]
            n12 = nbr_ref[n1, d2]
            n13 = nbr_ref[n1, d3]
            n23 = nbr_ref[n2, d3]
            n123 = nbr_ref[n12, d3]
            ids.append(dict(
                partners=(n1, n2, n3),
                p1=[n1], p2=[n2, n12], p3=[n3, n13, n23, n123],
            ))

        barrier_sem = pltpu.get_barrier_semaphore()
        for d in range(3):
            pl.semaphore_signal(
                barrier_sem, inc=1,
                device_id=(nbr_ref[my, d],),
                device_id_type=pl.DeviceIdType.MESH,
            )
        pl.semaphore_wait(barrier_sem, 3)

        scale = sx_ref[0] * sw_ref[0]

        out_copies = []

        def compute(rows, row0):
            acc = lax.dot_general(
                rows, wv_ref[...],
                (((1,), (0,)), ((), ())),
                preferred_element_type=jnp.int32,
            )
            y = jnp.maximum(acc.astype(jnp.float32) * scale, 0.0)
            nrows = rows.shape[0]
            yv_ref[pl.ds(row0, nrows), :] = y
            cp = pltpu.make_async_copy(
                yv_ref.at[pl.ds(row0, nrows)],
                out_ref.at[pl.ds(row0, nrows)],
                out_sems.at[len(out_copies)],
            )
            cp.start()
            out_copies.append(cp)

        rdmas = [[None] * 7 for _ in range(3)]

        def start_phase(st, phase):
            off, sz = _STREAMS[st][1], _STREAMS[st][2]
            info = ids[st]
            partner = info["partners"][phase - 1]
            held = [None] + info["p1"] + info["p2"]
            n_send = 1 << (phase - 1)
            j0 = n_send - 1
            for i in range(n_send):
                origin = held[i]
                if origin is None:
                    src = (x_ref if phase == 1 else xv_ref).at[pl.ds(off, sz)]
                    dst = comm[st].at[my]
                else:
                    src = comm[st].at[origin]
                    dst = comm[st].at[origin]
                r = pltpu.make_async_remote_copy(
                    src_ref=src,
                    dst_ref=dst,
                    send_sem=send_sems.at[st, j0 + i],
                    recv_sem=recv_sems.at[st, j0 + i],
                    device_id=(partner,),
                    device_id_type=pl.DeviceIdType.MESH,
                )
                r.start()
                rdmas[st][j0 + i] = r

        for st in range(3):
            start_phase(st, 1)
        x_copy.wait()
        w_copy.wait()
        compute(xv_ref[...], my * m_per)

        for st in range(3):
            rdmas[st][0].wait_recv()
            start_phase(st, 2)
        for st in range(3):
            off = _STREAMS[st][1]
            o = ids[st]["p1"][0]
            compute(comm[st][o], o * m_per + off)

        for st in range(3):
            rdmas[st][1].wait_recv()
            rdmas[st][2].wait_recv()
            start_phase(st, 3)
        for st in range(3):
            off = _STREAMS[st][1]
            for o in ids[st]["p2"]:
                compute(comm[st][o], o * m_per + off)

        for j in range(4):
            for st in range(3):
                rdmas[st][3 + j].wait_recv()
                off = _STREAMS[st][1]
                o = ids[st]["p3"][j]
                compute(comm[st][o], o * m_per + off)

        for st in range(3):
            for j in range(7):
                rdmas[st][j].wait_send()
        for cp in out_copies:
            cp.wait()

    return pl.pallas_call(
        body,
        out_shape=jax.ShapeDtypeStruct((M, n_per), jnp.float32),
        in_specs=[
            pl.BlockSpec(memory_space=pl.ANY),
            pl.BlockSpec(memory_space=pl.ANY),
            pl.BlockSpec(memory_space=pltpu.SMEM),
            pl.BlockSpec(memory_space=pltpu.SMEM),
            pl.BlockSpec(memory_space=pltpu.SMEM),
        ],
        out_specs=pl.BlockSpec(memory_space=pl.ANY),
        scratch_shapes=[
            pltpu.VMEM((N_DEV, _STREAMS[0][2], k), x.dtype),
            pltpu.VMEM((N_DEV, _STREAMS[1][2], k), x.dtype),
            pltpu.VMEM((N_DEV, _STREAMS[2][2], k), x.dtype),
            pltpu.SemaphoreType.DMA((3, 7)),
            pltpu.SemaphoreType.DMA((3, 7)),
            pltpu.VMEM((m_per, k), x.dtype),
            pltpu.VMEM((k, n_per), w_mat.dtype),
            pltpu.VMEM((M, n_per), jnp.float32),
            pltpu.SemaphoreType.DMA((2,)),
            pltpu.SemaphoreType.DMA((22,)),
        ],
        compiler_params=pltpu.CompilerParams(collective_id=0),
    )(x, w_mat, scale_x, scale_w, nbr_tab)


# device time: 69230 ns/iter; 1.1262x vs baseline; 1.1262x over previous
import numpy as np
import jax
import jax.numpy as jnp
from jax import lax
from jax.experimental import pallas as pl
from jax.experimental.pallas import tpu as pltpu

N_DEV = 8

_COORDS = {0: (0, 0, 0), 1: (1, 0, 0), 2: (1, 1, 0), 3: (0, 1, 0),
           4: (0, 0, 1), 5: (1, 0, 1), 6: (1, 1, 1), 7: (0, 1, 1)}
_INV = {v: k for k, v in _COORDS.items()}

_NBR = np.zeros((N_DEV, 3), dtype=np.int32)
for _u, (_x, _y, _z) in _COORDS.items():
    _NBR[_u, 0] = _INV[(1 - _x, _y, _z)]
    _NBR[_u, 1] = _INV[(_x, 1 - _y, _z)]
    _NBR[_u, 2] = _INV[(_x, _y, 1 - _z)]

_STREAMS = [
    ((0, 1, 2), 0, 192),
    ((1, 2, 0), 192, 160),
    ((2, 0, 1), 352, 160),
]


def kernel(x, w_mat, scale_x, scale_w):
    m_per, k = x.shape
    _, n_per = w_mat.shape
    M = N_DEV * m_per

    nbr_tab = jnp.asarray(_NBR)

    def body(x_ref, w_ref, sx_ref, sw_ref, nbr_ref, out_ref,
             c0_ref, c1_ref, c2_ref, send_sems, recv_sems):
        comm = [c0_ref, c1_ref, c2_ref]
        my = lax.axis_index("i")

        ids = []
        for (d1, d2, d3), _, _ in _STREAMS:
            n1 = nbr_ref[my, d1]
            n2 = nbr_ref[my, d2]
            n3 = nbr_ref[my, d3]
            n12 = nbr_ref[n1, d2]
            n13 = nbr_ref[n1, d3]
            n23 = nbr_ref[n2, d3]
            n123 = nbr_ref[n12, d3]
            ids.append(dict(
                partners=(n1, n2, n3),
                p1=[n1], p2=[n2, n12], p3=[n3, n13, n23, n123],
            ))

        barrier_sem = pltpu.get_barrier_semaphore()
        for d in range(3):
            pl.semaphore_signal(
                barrier_sem, inc=1,
                device_id=(nbr_ref[my, d],),
                device_id_type=pl.DeviceIdType.MESH,
            )
        pl.semaphore_wait(barrier_sem, 3)

        scale = sx_ref[0] * sw_ref[0]

        def compute(rows, row0):
            acc = lax.dot_general(
                rows, w_ref[...],
                (((1,), (0,)), ((), ())),
                preferred_element_type=jnp.int32,
            )
            y = jnp.maximum(acc.astype(jnp.float32) * scale, 0.0)
            out_ref[pl.ds(row0, rows.shape[0]), :] = y

        rdmas = [[None] * 7 for _ in range(3)]

        def issue(st, j, origin, partner):
            off, sz = _STREAMS[st][1], _STREAMS[st][2]
            if origin is None:
                src = x_ref.at[pl.ds(off, sz)]
                dst = comm[st].at[my]
            else:
                src = comm[st].at[origin]
                dst = comm[st].at[origin]
            r = pltpu.make_async_remote_copy(
                src_ref=src,
                dst_ref=dst,
                send_sem=send_sems.at[st, j],
                recv_sem=recv_sems.at[st, j],
                device_id=(partner,),
                device_id_type=pl.DeviceIdType.MESH,
            )
            r.start()
            rdmas[st][j] = r

        for st in range(3):
            p1, p2, p3 = ids[st]["partners"]
            issue(st, 0, None, p1)
            issue(st, 1, None, p2)
            issue(st, 3, None, p3)
        compute(x_ref[...], my * m_per)

        for st in range(3):
            rdmas[st][0].wait_recv()
            n1 = ids[st]["p1"][0]
            issue(st, 2, n1, ids[st]["partners"][1])
            issue(st, 4, n1, ids[st]["partners"][2])
        for st in range(3):
            off = _STREAMS[st][1]
            o = ids[st]["p1"][0]
            compute(comm[st][o], o * m_per + off)

        for st in range(3):
            rdmas[st][1].wait_recv()
            n2 = ids[st]["p2"][0]
            issue(st, 5, n2, ids[st]["partners"][2])
        for st in range(3):
            off = _STREAMS[st][1]
            o = ids[st]["p2"][0]
            compute(comm[st][o], o * m_per + off)

        for st in range(3):
            rdmas[st][2].wait_recv()
            n12 = ids[st]["p2"][1]
            issue(st, 6, n12, ids[st]["partners"][2])
        for st in range(3):
            off = _STREAMS[st][1]
            o = ids[st]["p2"][1]
            compute(comm[st][o], o * m_per + off)

        for j in (3, 4, 5, 6):
            for st in range(3):
                rdmas[st][j].wait_recv()
                off = _STREAMS[st][1]
                o = ids[st]["p3"][j - 3]
                compute(comm[st][o], o * m_per + off)

        for st in range(3):
            for j in range(7):
                rdmas[st][j].wait_send()

    return pl.pallas_call(
        body,
        out_shape=jax.ShapeDtypeStruct((M, n_per), jnp.float32),
        in_specs=[
            pl.BlockSpec(memory_space=pltpu.VMEM),
            pl.BlockSpec(memory_space=pltpu.VMEM),
            pl.BlockSpec(memory_space=pltpu.SMEM),
            pl.BlockSpec(memory_space=pltpu.SMEM),
            pl.BlockSpec(memory_space=pltpu.SMEM),
        ],
        out_specs=pl.BlockSpec(memory_space=pltpu.VMEM),
        scratch_shapes=[
            pltpu.VMEM((N_DEV, _STREAMS[0][2], k), x.dtype),
            pltpu.VMEM((N_DEV, _STREAMS[1][2], k), x.dtype),
            pltpu.VMEM((N_DEV, _STREAMS[2][2], k), x.dtype),
            pltpu.SemaphoreType.DMA((3, 7)),
            pltpu.SemaphoreType.DMA((3, 7)),
        ],
        compiler_params=pltpu.CompilerParams(collective_id=0),
    )(x, w_mat, scale_x, scale_w, nbr_tab)
